# baseline (speedup 1.0000x reference)
import numpy as np
import jax
import jax.numpy as jnp
from jax.experimental import pallas as pl


def _gather_idx():
    o = np.arange(512)
    ch = o // 64
    t = o % 64
    s = (t // 16) * 4 + (t % 8) // 2
    return jnp.asarray(ch * 16 + s, dtype=jnp.int32)


def _body(idx_ref, x_ref, o_ref):
    x = x_ref[...]
    idx = idx_ref[0, :]
    idx3 = jnp.broadcast_to(idx[None, None, :], x.shape[:2] + (512,))
    o_ref[...] = jnp.take_along_axis(x, idx3, axis=2)


def kernel(arg0_1):
    B = arg0_1.shape[0]
    bm = 8
    xf = arg0_1.reshape(B, 96, 128)
    idx = _gather_idx().reshape(1, 512)
    out = pl.pallas_call(
        _body,
        grid=(B // bm,),
        in_specs=[
            pl.BlockSpec((1, 512), lambda i: (0, 0)),
            pl.BlockSpec((bm, 96, 128), lambda i: (i, 0, 0)),
        ],
        out_specs=pl.BlockSpec((bm, 96, 512), lambda i: (i, 0, 0)),
        out_shape=jax.ShapeDtypeStruct((B, 96, 512), jnp.float32),
    )(idx, xf)
    return (out.reshape(B, 768, 8, 8),)


# speedup vs baseline: 1.0381x; 1.0381x over previous
import numpy as np
import jax
import jax.numpy as jnp
from jax.experimental import pallas as pl


def _gather_idx():
    o = np.arange(512)
    ch = o // 64
    t = o % 64
    s = (t // 16) * 4 + (t % 8) // 2
    return jnp.asarray(ch * 16 + s, dtype=jnp.int32)


def _body(idx_ref, x_ref, o_ref):
    x = x_ref[...]
    idx = idx_ref[0, :]
    idx3 = jnp.broadcast_to(idx[None, None, :], x.shape[:2] + (512,))
    o_ref[...] = jnp.take_along_axis(x, idx3, axis=2)


def kernel(arg0_1):
    B = arg0_1.shape[0]
    bm = 32
    xf = arg0_1.reshape(B, 96, 128)
    idx = _gather_idx().reshape(1, 512)
    out = pl.pallas_call(
        _body,
        grid=(B // bm,),
        in_specs=[
            pl.BlockSpec((1, 512), lambda i: (0, 0)),
            pl.BlockSpec((bm, 96, 128), lambda i: (i, 0, 0)),
        ],
        out_specs=pl.BlockSpec((bm, 96, 512), lambda i: (i, 0, 0)),
        out_shape=jax.ShapeDtypeStruct((B, 96, 512), jnp.float32),
    )(idx, xf)
    return (out.reshape(B, 768, 8, 8),)


# speedup vs baseline: 6.2038x; 5.9764x over previous
import jax
import jax.numpy as jnp
from jax.experimental import pallas as pl


def _body(x_ref, o_ref):
    x = x_ref[...]
    y = jnp.repeat(x, 2, axis=1)
    o_ref[...] = jnp.repeat(y, 2, axis=0)


def kernel(arg0_1):
    B = arg0_1.shape[0]
    bm = 64
    xt = jnp.transpose(arg0_1, (0, 2, 3, 1)).reshape(B * 4, 4, 768)
    out = pl.pallas_call(
        _body,
        grid=(B * 4 // bm,),
        in_specs=[pl.BlockSpec((bm, 4, 768), lambda i: (i, 0, 0))],
        out_specs=pl.BlockSpec((2 * bm, 8, 768), lambda i: (i, 0, 0)),
        out_shape=jax.ShapeDtypeStruct((B * 8, 8, 768), jnp.float32),
    )(xt)
    return (jnp.transpose(out.reshape(B, 8, 8, 768), (0, 3, 1, 2)),)


# speedup vs baseline: 7.4490x; 1.2007x over previous
import jax
import jax.numpy as jnp
from jax.experimental import pallas as pl


def _body(x_ref, o_ref):
    x = x_ref[...]
    y = jnp.repeat(x, 2, axis=1)
    o_ref[...] = jnp.repeat(y, 2, axis=0)


def kernel(arg0_1):
    B = arg0_1.shape[0]
    bm = 128
    xt = jnp.transpose(arg0_1, (0, 2, 3, 1)).reshape(B * 4, 4, 768)
    out = pl.pallas_call(
        _body,
        grid=(B * 4 // bm,),
        in_specs=[pl.BlockSpec((bm, 4, 768), lambda i: (i, 0, 0))],
        out_specs=pl.BlockSpec((2 * bm, 8, 768), lambda i: (i, 0, 0)),
        out_shape=jax.ShapeDtypeStruct((B * 8, 8, 768), jnp.float32),
    )(xt)
    return (jnp.transpose(out.reshape(B, 8, 8, 768), (0, 3, 1, 2)),)


# speedup vs baseline: 7.9732x; 1.0704x over previous
import jax
import jax.numpy as jnp
from jax.experimental import pallas as pl


def _body(x_ref, o_ref):
    x = x_ref[...]
    bm = x.shape[0]
    idx = jax.lax.broadcasted_iota(jnp.int32, (bm, 8, 768), 1) // 2
    y = jnp.take_along_axis(x, idx, axis=1)
    o_ref[...] = jnp.repeat(y, 2, axis=0)


def kernel(arg0_1):
    B = arg0_1.shape[0]
    bm = 128
    xt = jnp.transpose(arg0_1, (0, 2, 3, 1)).reshape(B * 4, 4, 768)
    out = pl.pallas_call(
        _body,
        grid=(B * 4 // bm,),
        in_specs=[pl.BlockSpec((bm, 4, 768), lambda i: (i, 0, 0))],
        out_specs=pl.BlockSpec((2 * bm, 8, 768), lambda i: (i, 0, 0)),
        out_shape=jax.ShapeDtypeStruct((B * 8, 8, 768), jnp.float32),
    )(xt)
    return (jnp.transpose(out.reshape(B, 8, 8, 768), (0, 3, 1, 2)),)
